# baseline (speedup 1.0000x reference)
import jax
import jax.numpy as jnp
from jax import lax
from jax.experimental import pallas as pl
from jax.experimental.pallas import tpu as pltpu
from jax.experimental.pallas import tpu_sc as plsc

_Q = 8000
_S = 2000
_N = 10000
_EMB = 128
_E = 320000

_AUG = 144
_NSC = 2
_NTILE = 16
_NW = _NSC * _NTILE
_CHUNK = 128
_EPT = 10240
_NCHUNK = _EPT // _CHUNK
_EPAD = _NW * _EPT
_AGG_ROWS = 8192
_RPT = _AGG_ROWS // _NTILE


def _sc_body(nodes_hbm, src_hbm, dst_hbm, out_hbm,
             sidx, didx, rows_a, rows_b, gs_a, gs_b, ss_a, ss_b, agg_sh):
    cid = lax.axis_index("c")
    sid = lax.axis_index("s")
    w = cid * _NTILE + sid

    pltpu.sync_copy(src_hbm.at[w], sidx)
    pltpu.sync_copy(dst_hbm.at[w], didx)

    def _zrow(r, carry):
        for k in range(_AUG // 16):
            rows_a[r, pl.ds(k * 16, 16)] = jnp.zeros((16,), jnp.float32)
        return carry

    lax.fori_loop(0, _CHUNK, _zrow, 0)
    for k in range(_RPT // _CHUNK):
        pltpu.sync_copy(rows_a, agg_sh.at[pl.ds(sid * _RPT + k * _CHUNK, _CHUNK)])
    plsc.subcore_barrier()

    def _gather(c, buf, sem):
        pltpu.async_copy(nodes_hbm.at[didx.at[c]], buf, sem)

    def _gather_wait(c, buf, sem):
        pltpu.make_async_copy(nodes_hbm.at[didx.at[c]], buf, sem).wait()

    def _scatter(c, buf, sem):
        pltpu.async_copy(buf, agg_sh.at[sidx.at[c]], sem, add=True)

    def _scatter_wait(c, buf, sem):
        pltpu.make_async_copy(buf, agg_sh.at[sidx.at[c]], sem).wait()

    _gather(0, rows_a, gs_a)
    _gather(1, rows_b, gs_b)
    _gather_wait(0, rows_a, gs_a)
    _scatter(0, rows_a, ss_a)
    _gather_wait(1, rows_b, gs_b)
    _scatter(1, rows_b, ss_b)

    def _pipe(i, carry):
        c = 2 + 2 * i
        _scatter_wait(c - 2, rows_a, ss_a)
        _gather(c, rows_a, gs_a)
        _scatter_wait(c - 1, rows_b, ss_b)
        _gather(c + 1, rows_b, gs_b)
        _gather_wait(c, rows_a, gs_a)
        _scatter(c, rows_a, ss_a)
        _gather_wait(c + 1, rows_b, gs_b)
        _scatter(c + 1, rows_b, ss_b)
        return carry

    lax.fori_loop(0, (_NCHUNK - 2) // 2, _pipe, 0)
    _scatter_wait(_NCHUNK - 2, rows_a, ss_a)
    _scatter_wait(_NCHUNK - 1, rows_b, ss_b)
    plsc.subcore_barrier()

    for k in range(_RPT // _CHUNK):
        base = sid * _RPT + k * _CHUNK
        pltpu.sync_copy(agg_sh.at[pl.ds(base, _CHUNK)], rows_a)
        pltpu.sync_copy(rows_a, out_hbm.at[cid, pl.ds(base, _CHUNK)])


_sc_aggregate = pl.kernel(
    _sc_body,
    out_type=jax.ShapeDtypeStruct((_NSC, _AGG_ROWS, _AUG), jnp.float32),
    mesh=plsc.VectorSubcoreMesh(core_axis_name="c", subcore_axis_name="s"),
    scratch_types=[
        pltpu.VMEM((_NCHUNK, _CHUNK), jnp.int32),
        pltpu.VMEM((_NCHUNK, _CHUNK), jnp.int32),
        pltpu.VMEM((_CHUNK, _AUG), jnp.float32),
        pltpu.VMEM((_CHUNK, _AUG), jnp.float32),
        pltpu.SemaphoreType.DMA,
        pltpu.SemaphoreType.DMA,
        pltpu.SemaphoreType.DMA,
        pltpu.SemaphoreType.DMA,
        pltpu.VMEM_SHARED((_AGG_ROWS, _AUG), jnp.float32),
    ],
    compiler_params=pltpu.CompilerParams(use_tc_tiling_on_sc=False),
)

_BLK = 1000


def _head_body(part_ref, nodes_ref, w1_ref, w2_ref, bias_ref, out_ref):
    s = part_ref[0] + part_ref[1]
    agg = s[:, :_EMB]
    deg = s[:, _EMB:_EMB + 1]
    aggn = agg / jnp.maximum(deg, 1.0)
    h = jnp.dot(aggn, w1_ref[...], preferred_element_type=jnp.float32)
    h = h + jnp.dot(nodes_ref[...], w2_ref[...], preferred_element_type=jnp.float32)
    h = jnp.maximum(h, 0.0)
    out_ref[0] = h + bias_ref[0:1, :]
    out_ref[1] = h + bias_ref[1:2, :]


_head = pl.pallas_call(
    _head_body,
    grid=(_Q // _BLK,),
    in_specs=[
        pl.BlockSpec((_NSC, _BLK, _AUG), lambda i: (0, i, 0)),
        pl.BlockSpec((_BLK, _EMB), lambda i: (i, 0)),
        pl.BlockSpec((_EMB, _EMB), lambda i: (0, 0)),
        pl.BlockSpec((_EMB, _EMB), lambda i: (0, 0)),
        pl.BlockSpec((8, _EMB), lambda i: (0, 0)),
    ],
    out_specs=pl.BlockSpec((2, _BLK, _EMB), lambda i: (0, i, 0)),
    out_shape=jax.ShapeDtypeStruct((2, _Q, _EMB), jnp.float32),
)


def kernel(nodes_features, edge_index, W1, W2, correct_bias, incorrect_bias):
    nf = nodes_features.astype(jnp.float32)
    src = edge_index[0].astype(jnp.int32)
    dst = edge_index[1].astype(jnp.int32)

    nodes_aug = jnp.concatenate(
        [nf,
         jnp.ones((_N, 1), jnp.float32),
         jnp.zeros((_N, _AUG - _EMB - 1), jnp.float32)], axis=1)

    npad = _EPAD - _E
    pad_src = _Q + (jnp.arange(npad, dtype=jnp.int32) % (_AGG_ROWS - _Q))
    src_p = jnp.concatenate([src, pad_src]).reshape(_NW, _NCHUNK, _CHUNK)
    dst_p = jnp.concatenate(
        [dst, jnp.zeros((npad,), jnp.int32)]).reshape(_NW, _NCHUNK, _CHUNK)

    partials = _sc_aggregate(nodes_aug, src_p, dst_p)

    bias2 = jnp.concatenate(
        [incorrect_bias.astype(jnp.float32),
         correct_bias.astype(jnp.float32),
         jnp.zeros((6, _EMB), jnp.float32)], axis=0)

    halves = _head(partials, nf, W1.astype(jnp.float32),
                   W2.astype(jnp.float32), bias2)

    return jnp.concatenate(
        [halves.reshape(2 * _Q, _EMB), jnp.zeros((1, _EMB), jnp.float32)], axis=0)


# speedup vs baseline: 2.3460x; 2.3460x over previous
import jax
import jax.numpy as jnp
from jax import lax
from jax.experimental import pallas as pl
from jax.experimental.pallas import tpu as pltpu
from jax.experimental.pallas import tpu_sc as plsc

_Q = 8000
_S = 2000
_N = 10000
_EMB = 128
_E = 320000

_AUG = 144
_NSC = 2
_NTILE = 16
_NW = _NSC * _NTILE
_CHUNK = 64
_NCHUNK = -(-_E // (_NW * _CHUNK))
_NCHUNK += _NCHUNK % 2
_EPT = _NCHUNK * _CHUNK
_EPAD = _NW * _EPT
_AGG_ROWS = 8192
_RPT = _AGG_ROWS // _NTILE
_SKL_ROWS = 2048
_SPT = _SKL_ROWS // _NTILE
_TBL_ROWS = _Q + _SKL_ROWS


def _sc_body(nodes_hbm, src_hbm, dst_hbm, out_hbm,
             sidx, didx, rows_a, rows_b, gs_a, gs_b, ss_a, ss_b, agg_sh):
    cid = lax.axis_index("c")
    sid = lax.axis_index("s")
    w = cid * _NTILE + sid

    pltpu.sync_copy(src_hbm.at[w], sidx)
    pltpu.sync_copy(dst_hbm.at[w], didx)

    def _rebase(r, carry):
        for k in range(_CHUNK // 16):
            sl = pl.ds(k * 16, 16)
            didx[r, sl] = didx[r, sl] + (_AGG_ROWS - _Q)
        return carry

    lax.fori_loop(0, _NCHUNK, _rebase, 0)

    def _zrow(r, carry):
        for k in range(_AUG // 16):
            rows_a[r, pl.ds(k * 16, 16)] = jnp.zeros((16,), jnp.float32)
        return carry

    lax.fori_loop(0, _CHUNK, _zrow, 0)
    for k in range(_RPT // _CHUNK):
        pltpu.sync_copy(rows_a, agg_sh.at[pl.ds(sid * _RPT + k * _CHUNK, _CHUNK)])

    for k in range(_SPT // _CHUNK):
        pltpu.sync_copy(nodes_hbm.at[pl.ds(_Q + sid * _SPT + k * _CHUNK, _CHUNK)],
                        rows_a)
        pltpu.sync_copy(
            rows_a,
            agg_sh.at[pl.ds(_AGG_ROWS + sid * _SPT + k * _CHUNK, _CHUNK)])
    plsc.subcore_barrier()

    def _gather(c, buf, sem):
        pltpu.async_copy(agg_sh.at[didx.at[c]], buf, sem)

    def _gather_wait(c, buf, sem):
        pltpu.make_async_copy(agg_sh.at[didx.at[c]], buf, sem).wait()

    def _scatter(c, buf, sem):
        pltpu.async_copy(buf, agg_sh.at[sidx.at[c]], sem, add=True)

    def _scatter_wait(c, buf, sem):
        pltpu.make_async_copy(buf, agg_sh.at[sidx.at[c]], sem).wait()

    _gather(0, rows_a, gs_a)
    _gather(1, rows_b, gs_b)
    _gather_wait(0, rows_a, gs_a)
    _scatter(0, rows_a, ss_a)
    _gather_wait(1, rows_b, gs_b)
    _scatter(1, rows_b, ss_b)

    def _pipe(i, carry):
        c = 2 + 2 * i
        _scatter_wait(c - 2, rows_a, ss_a)
        _gather(c, rows_a, gs_a)
        _scatter_wait(c - 1, rows_b, ss_b)
        _gather(c + 1, rows_b, gs_b)
        _gather_wait(c, rows_a, gs_a)
        _scatter(c, rows_a, ss_a)
        _gather_wait(c + 1, rows_b, gs_b)
        _scatter(c + 1, rows_b, ss_b)
        return carry

    lax.fori_loop(0, (_NCHUNK - 2) // 2, _pipe, 0)
    _scatter_wait(_NCHUNK - 2, rows_a, ss_a)
    _scatter_wait(_NCHUNK - 1, rows_b, ss_b)
    plsc.subcore_barrier()

    for k in range(_RPT // _CHUNK):
        base = sid * _RPT + k * _CHUNK
        pltpu.sync_copy(agg_sh.at[pl.ds(base, _CHUNK)], rows_a)
        pltpu.sync_copy(rows_a, out_hbm.at[cid, pl.ds(base, _CHUNK)])


_sc_aggregate = pl.kernel(
    _sc_body,
    out_type=jax.ShapeDtypeStruct((_NSC, _AGG_ROWS, _AUG), jnp.float32),
    mesh=plsc.VectorSubcoreMesh(core_axis_name="c", subcore_axis_name="s",
                                num_cores=_NSC),
    scratch_types=[
        pltpu.VMEM((_NCHUNK, _CHUNK), jnp.int32),
        pltpu.VMEM((_NCHUNK, _CHUNK), jnp.int32),
        pltpu.VMEM((_CHUNK, _AUG), jnp.float32),
        pltpu.VMEM((_CHUNK, _AUG), jnp.float32),
        pltpu.SemaphoreType.DMA,
        pltpu.SemaphoreType.DMA,
        pltpu.SemaphoreType.DMA,
        pltpu.SemaphoreType.DMA,
        pltpu.VMEM_SHARED((_AGG_ROWS + _SKL_ROWS, _AUG), jnp.float32),
    ],
    compiler_params=pltpu.CompilerParams(use_tc_tiling_on_sc=False),
)

_BLK = 1000


def _head_body(part_ref, nodes_ref, w1_ref, w2_ref, bias_ref, out_ref):
    s = part_ref[0]
    for p in range(1, _NSC):
        s = s + part_ref[p]
    agg = s[:, :_EMB]
    deg = s[:, _EMB:_EMB + 1]
    aggn = agg / jnp.maximum(deg, 1.0)
    h = jnp.dot(aggn, w1_ref[...], preferred_element_type=jnp.float32)
    h = h + jnp.dot(nodes_ref[...], w2_ref[...], preferred_element_type=jnp.float32)
    h = jnp.maximum(h, 0.0)
    out_ref[0] = h + bias_ref[0:1, :]
    out_ref[1] = h + bias_ref[1:2, :]


_head = pl.pallas_call(
    _head_body,
    grid=(_Q // _BLK,),
    in_specs=[
        pl.BlockSpec((_NSC, _BLK, _AUG), lambda i: (0, i, 0)),
        pl.BlockSpec((_BLK, _EMB), lambda i: (i, 0)),
        pl.BlockSpec((_EMB, _EMB), lambda i: (0, 0)),
        pl.BlockSpec((_EMB, _EMB), lambda i: (0, 0)),
        pl.BlockSpec((8, _EMB), lambda i: (0, 0)),
    ],
    out_specs=pl.BlockSpec((2, _BLK, _EMB), lambda i: (0, i, 0)),
    out_shape=jax.ShapeDtypeStruct((2, _Q, _EMB), jnp.float32),
)


def kernel(nodes_features, edge_index, W1, W2, correct_bias, incorrect_bias):
    nf = nodes_features.astype(jnp.float32)
    src = edge_index[0].astype(jnp.int32)
    dst = edge_index[1].astype(jnp.int32)

    nodes_aug = jnp.concatenate(
        [nf,
         jnp.ones((_N, 1), jnp.float32),
         jnp.zeros((_N, _AUG - _EMB - 1), jnp.float32)], axis=1)
    nodes_aug = jnp.concatenate(
        [nodes_aug, jnp.zeros((_TBL_ROWS - _N, _AUG), jnp.float32)], axis=0)

    npad = _EPAD - _E
    pad_src = _Q + (jnp.arange(npad, dtype=jnp.int32) % (_AGG_ROWS - _Q))
    src_p = jnp.concatenate([src, pad_src]).reshape(_NW, _NCHUNK, _CHUNK)
    dst_p = jnp.concatenate(
        [dst, jnp.full((npad,), _Q, jnp.int32)]).reshape(_NW, _NCHUNK, _CHUNK)

    partials = _sc_aggregate(nodes_aug, src_p, dst_p)

    bias2 = jnp.concatenate(
        [incorrect_bias.astype(jnp.float32),
         correct_bias.astype(jnp.float32),
         jnp.zeros((6, _EMB), jnp.float32)], axis=0)

    halves = _head(partials, nf, W1.astype(jnp.float32),
                   W2.astype(jnp.float32), bias2)

    return jnp.concatenate(
        [halves.reshape(2 * _Q, _EMB), jnp.zeros((1, _EMB), jnp.float32)], axis=0)
